# Initial kernel scaffold; baseline (speedup 1.0000x reference)
import jax
import jax.numpy as jnp
from jax.experimental import pallas as pl


def kernel(indices, table, proj):
    raise NotImplementedError("write your pallas kernel here")


# speedup vs baseline: 16.3857x; 16.3857x over previous
import functools

import jax
import jax.numpy as jnp
from jax import lax
from jax.experimental import pallas as pl
from jax.experimental.pallas import tpu as pltpu
from jax.experimental.pallas import tpu_sc as plsc

_B = 16384
_F = 26
_D = 16
_N = _B * _F
_NW = 32
_PER_W = _N // _NW
_CHUNK = 128
_GROUP = 8
_ROWS_PER_GROUP = _CHUNK * _GROUP
_NGROUPS = _PER_W // _ROWS_PER_GROUP
_NCHUNKS = _PER_W // _CHUNK


def _sc_gather(table, idx2d):
    mesh = plsc.VectorSubcoreMesh(core_axis_name="c", subcore_axis_name="s")

    @functools.partial(
        pl.kernel,
        out_type=jax.ShapeDtypeStruct((_N, _D), jnp.float32),
        mesh=mesh,
        scratch_types=[
            pltpu.VMEM((_NCHUNKS, _CHUNK), jnp.int32),
            pltpu.VMEM((_ROWS_PER_GROUP, _D), jnp.float32),
            pltpu.SemaphoreType.DMA,
        ],
        compiler_params=pltpu.CompilerParams(use_tc_tiling_on_sc=False),
    )
    def k(table_hbm, idx_hbm, out_hbm, idx_v, rows_v, sem):
        wid = lax.axis_index("s") * 2 + lax.axis_index("c")
        base = wid * _PER_W
        pltpu.sync_copy(idx_hbm.at[wid], idx_v)

        def group_body(g, carry):
            copies = []
            for c in range(_GROUP):
                cp = pltpu.async_copy(
                    table_hbm.at[idx_v.at[g * _GROUP + c]],
                    rows_v.at[pl.ds(c * _CHUNK, _CHUNK)],
                    sem,
                )
                copies.append(cp)
            for cp in copies:
                cp.wait()
            pltpu.sync_copy(
                rows_v,
                out_hbm.at[pl.ds(base + g * _ROWS_PER_GROUP, _ROWS_PER_GROUP)],
            )
            return carry

        lax.fori_loop(0, _NGROUPS, group_body, 0)

    return k(table, idx2d)


def _tc_project(flat, proj):
    bm = 1024

    def body(x_ref, p_ref, o_ref):
        acc = jnp.dot(x_ref[...], p_ref[...],
                      preferred_element_type=jnp.float32)
        nrm = jnp.sqrt(jnp.sum(acc * acc, axis=-1, keepdims=True))
        o_ref[...] = acc / (nrm + 1e-12)

    return pl.pallas_call(
        body,
        grid=(_B // bm,),
        in_specs=[
            pl.BlockSpec((bm, _F * _D), lambda i: (i, 0)),
            pl.BlockSpec((_F * _D, 64), lambda i: (0, 0)),
        ],
        out_specs=pl.BlockSpec((bm, 64), lambda i: (i, 0)),
        out_shape=jax.ShapeDtypeStruct((_B, 64), jnp.float32),
    )(flat, proj)


@jax.jit
def kernel(indices, table, proj):
    idx = indices.astype(jnp.int32).reshape(_NW, _NCHUNKS, _CHUNK)
    rows = _sc_gather(table, idx)
    flat = rows.reshape(_B, _F * _D)
    return _tc_project(flat, proj)


# speedup vs baseline: 44.0078x; 2.6857x over previous
import functools

import jax
import jax.numpy as jnp
from jax import lax
from jax.experimental import pallas as pl
from jax.experimental.pallas import tpu as pltpu
from jax.experimental.pallas import tpu_sc as plsc

_B = 16384
_F = 26
_D = 16
_N = _B * _F
_NW = 32
_PER_W = _N // _NW
_CHUNK = 128
_GROUP = 8
_ROWS_PER_GROUP = _CHUNK * _GROUP
_NGROUPS = _PER_W // _ROWS_PER_GROUP
_NCHUNKS = _PER_W // _CHUNK


_NFULL = 976
_TAIL_ROWS = 576

_PERM_DNUMS = lax.GatherDimensionNumbers(
    offset_dims=(), collapsed_slice_dims=(0,), start_index_map=(0,))


def _vperm(x, idx):
    return lax.gather(x, idx[:, None], _PERM_DNUMS, (1,),
                      mode=lax.GatherScatterMode.PROMISE_IN_BOUNDS)


def _sc_detile(natT, tail_rows):
    mesh = plsc.VectorSubcoreMesh(core_axis_name="c", subcore_axis_name="s")

    @functools.partial(
        pl.kernel,
        out_type=jax.ShapeDtypeStruct((16000000,), jnp.float32),
        mesh=mesh,
        scratch_types=[
            pltpu.VMEM((2, 16, 1024), jnp.float32),
            pltpu.VMEM((32768,), jnp.float32),
            pltpu.SemaphoreType.DMA((2,)),
            pltpu.SemaphoreType.DMA((2,)),
        ],
        compiler_params=pltpu.CompilerParams(needs_layout_passes=False),
    )
    def k(nat_hbm, tail_hbm, out_hbm, vin, vout, sem_in, sem_out):
        wid = lax.axis_index("s") * 2 + lax.axis_index("c")
        cnt = jnp.where(wid < _NFULL % _NW, _NFULL // _NW + 1, _NFULL // _NW)
        iota = jax.lax.iota(jnp.int32, 16)
        scat = iota * 16

        def in_copy(g, grp):
            base = (wid + g * _NW) * 1024
            return (nat_hbm.at[:, pl.ds(base, 1024)], vin.at[grp],
                    sem_in.at[grp])

        def fire_in(g, grp):
            pltpu.async_copy(*in_copy(g, grp))

        def drain_in(g, grp):
            pltpu.make_async_copy(*in_copy(g, grp)).wait()

        def out_copy(g, grp):
            return (vout.at[pl.ds(grp * 16384, 16384)],
                    out_hbm.at[pl.ds((wid + g * _NW) * 16384, 16384)],
                    sem_out.at[grp])

        def transpose_block(grp, lbase, obase):
            for l0 in range(0, 128, 16):
                sv = scat + (obase + l0 * 16)
                vecs = [vin[grp, d, pl.ds(lbase + l0, 16)] for d in range(_D)]
                for d in range(_D):
                    plsc.store_scatter(vout, [sv + d], vecs[d])

        def chunk_body(g, carry):
            grp = g & 1

            @pl.when(g + 1 < cnt)
            def _prefetch():
                fire_in(g + 1, (g + 1) & 1)

            drain_in(g, grp)

            @pl.when(g >= 2)
            def _reclaim():
                pltpu.make_async_copy(*out_copy(g - 2, grp)).wait()

            for i in range(8):
                transpose_block(grp, i * 128, grp * 16384 + i * 2048)
            pltpu.async_copy(*out_copy(g, grp))
            return carry

        fire_in(0, 0)
        lax.fori_loop(0, cnt, chunk_body, 0)
        pltpu.make_async_copy(*out_copy(cnt - 2, (cnt - 2) & 1)).wait()
        pltpu.make_async_copy(*out_copy(cnt - 1, (cnt - 1) & 1)).wait()

        @pl.when(wid == _NW - 1)
        def _tail():
            base = _NFULL * 1024
            w1 = pltpu.async_copy(
                nat_hbm.at[:, pl.ds(base, 512)],
                vin.at[0, :, pl.ds(0, 512)], sem_in.at[0],
            )
            w2 = pltpu.async_copy(
                tail_hbm, vin.at[0, pl.ds(0, 8), pl.ds(512, 128)],
                sem_in.at[0],
            )
            w1.wait()
            w2.wait()
            for i in range(4):
                transpose_block(0, i * 128, i * 2048)
            for r in range(8):
                for c0 in range(0, 128, 16):
                    vout[pl.ds(8192 + r * 128 + c0, 16)] = (
                        vin[0, r, pl.ds(512 + c0, 16)])
            pltpu.async_copy(
                vout.at[pl.ds(0, 9216)],
                out_hbm.at[pl.ds(_NFULL * 16384, 9216)], sem_out.at[0],
            ).wait()

    return k(natT, tail_rows)


def _sc_gather(table, idx2d):
    mesh = plsc.VectorSubcoreMesh(core_axis_name="c", subcore_axis_name="s")

    @functools.partial(
        pl.kernel,
        out_type=jax.ShapeDtypeStruct((_N, _D), jnp.float32),
        mesh=mesh,
        scratch_types=[
            pltpu.VMEM((_NCHUNKS, _CHUNK), jnp.int32),
            pltpu.VMEM((3, _ROWS_PER_GROUP, _D), jnp.float32),
            pltpu.SemaphoreType.DMA((3,)),
            pltpu.SemaphoreType.DMA((3,)),
        ],
        compiler_params=pltpu.CompilerParams(use_tc_tiling_on_sc=False),
    )
    def k(table_hbm, idx_hbm, out_hbm, idx_v, rows_v, sem_g, sem_w):
        wid = lax.axis_index("s") * 2 + lax.axis_index("c")
        base = wid * _PER_W
        pltpu.sync_copy(idx_hbm.at[wid], idx_v)

        def gathers(g, grp):
            out = []
            for c in range(_GROUP):
                out.append((
                    table_hbm.at[idx_v.at[g * _GROUP + c]],
                    rows_v.at[grp, pl.ds(c * _CHUNK, _CHUNK)],
                    sem_g.at[grp],
                ))
            return out

        def out_copy(g, grp):
            return (rows_v.at[grp],
                    out_hbm.at[pl.ds(base + g * _ROWS_PER_GROUP,
                                     _ROWS_PER_GROUP)],
                    sem_w.at[grp])

        def group_body(g, carry):
            grp = lax.rem(g, 3)

            @pl.when(g >= 2)
            def _reclaim():
                pltpu.make_async_copy(*out_copy(g - 2, lax.rem(g - 2, 3))).wait()

            @pl.when(g + 1 < _NGROUPS)
            def _prefetch():
                for args in gathers(g + 1, lax.rem(g + 1, 3)):
                    pltpu.async_copy(*args)

            for args in gathers(g, grp):
                pltpu.make_async_copy(*args).wait()

            pltpu.async_copy(*out_copy(g, grp))
            return carry

        for args in gathers(0, 0):
            pltpu.async_copy(*args)
        lax.fori_loop(0, _NGROUPS, group_body, 0)
        pltpu.make_async_copy(*out_copy(_NGROUPS - 2, (_NGROUPS - 2) % 3)).wait()
        pltpu.make_async_copy(*out_copy(_NGROUPS - 1, (_NGROUPS - 1) % 3)).wait()

    return k(table, idx2d)


def _tc_project(flat, proj):
    bm = 1024

    def body(x_ref, p_ref, o_ref):
        acc = jnp.dot(x_ref[...], p_ref[...],
                      preferred_element_type=jnp.float32)
        nrm = jnp.sqrt(jnp.sum(acc * acc, axis=-1, keepdims=True))
        o_ref[...] = acc / (nrm + 1e-12)

    return pl.pallas_call(
        body,
        grid=(_B // bm,),
        in_specs=[
            pl.BlockSpec((bm, _F * _D), lambda i: (i, 0)),
            pl.BlockSpec((_F * _D, 64), lambda i: (0, 0)),
        ],
        out_specs=pl.BlockSpec((bm, 64), lambda i: (i, 0)),
        out_shape=jax.ShapeDtypeStruct((_B, 64), jnp.float32),
    )(flat, proj)


@jax.jit
def kernel(indices, table, proj):
    idx = indices.astype(jnp.int32).reshape(_NW, _NCHUNKS, _CHUNK)
    tail = table[999936:].reshape(8, 128)
    tbl = _sc_detile(table.T, tail).reshape(1000000, _D)
    rows = _sc_gather(tbl, idx)
    flat = rows.reshape(_B, _F * _D)
    return _tc_project(flat, proj)
